# speedup vs baseline: 3.9661x; 3.9661x over previous
import jax
import jax.numpy as jnp
from jax import lax
from jax.experimental import pallas as pl
from jax.experimental.pallas import tpu as pltpu

_N = 1000
_ROWS = 16384
_BR = 256
_K = 5


def _tc_body(lab_ref, x_ref, out_ref):
    x = x_ref[...]
    lab = lab_ref[...]
    iota = lax.broadcasted_iota(jnp.int32, (_BR, _N), 1)
    onehot = iota == lab
    true1 = jnp.sum(jnp.where(onehot, x, 0.0), axis=1, keepdims=True)
    m = jnp.where(onehot, -1e7, x)
    loss = jnp.zeros((_BR, 1), jnp.float32)
    for _ in range(_K):
        w = jnp.max(m, axis=1, keepdims=True)
        loss = loss + jnp.maximum(true1 - w + 1.0, 0.0)
        idx = jnp.min(jnp.where(m == w, iota, _N), axis=1, keepdims=True)
        m = jnp.where(iota == idx, -jnp.inf, m)
    out_ref[...] = loss


def kernel(logits, labels):
    lab2 = labels.astype(jnp.int32).reshape(_ROWS, 1)
    grid = (_ROWS // _BR,)
    out = pl.pallas_call(
        _tc_body,
        grid=grid,
        in_specs=[
            pl.BlockSpec((_BR, 1), lambda i: (i, 0)),
            pl.BlockSpec((_BR, _N), lambda i: (i, 0)),
        ],
        out_specs=pl.BlockSpec((_BR, 1), lambda i: (i, 0)),
        out_shape=jax.ShapeDtypeStruct((_ROWS, 1), jnp.float32),
    )(lab2, logits)
    return out.reshape(_ROWS)
